# baseline (speedup 1.0000x reference)
import functools

import jax
import jax.numpy as jnp
from jax import lax
from jax.experimental import pallas as pl
from jax.experimental.pallas import tpu as pltpu
from jax.experimental.pallas import tpu_sc as plsc

VOCAB_SIZE = 1000000
H = 64
B = 1024
L = 200
TOK = B * L
G = 128
SCALE = float(H) ** 0.5

_info = plsc.get_sparse_core_info()
NC = _info.num_cores
NS = _info.num_subcores
NW = NC * NS
ROWS_PER_W = B // NW
TOK_PER_W = ROWS_PER_W * L
GROUPS_PER_W = TOK_PER_W // G
NBUF = 5
ROUNDS = GROUPS_PER_W // NBUF


def _body(table_hbm, idx_hbm, out_hbm, idx_v, rows_v, gsem, osem, isem):
    wid = lax.axis_index("s") * NC + lax.axis_index("c")
    tok0 = wid * TOK_PER_W

    idx_dma = pltpu.make_async_copy(
        idx_hbm.at[pl.ds(tok0, TOK_PER_W)], idx_v, isem
    )
    idx_dma.start()
    idx_dma.wait()

    def gather_of(g, b):
        return pltpu.make_async_copy(
            table_hbm.at[idx_v.at[pl.ds(g * G, G)]], rows_v.at[b], gsem.at[b]
        )

    def write_of(g, b):
        return pltpu.make_async_copy(
            rows_v.at[b], out_hbm.at[pl.ds(tok0 + g * G, G)], osem.at[b]
        )

    for b in range(NBUF):
        gather_of(b, b).start()

    def round_(o, carry):
        for b in range(NBUF):
            g = o * NBUF + b
            gather_of(g, b).wait()
            write_of(g, b).start()
        for b in range(NBUF):
            g = o * NBUF + b
            write_of(g, b).wait()

            @pl.when(o < ROUNDS - 1)
            def _():
                gather_of(g + NBUF, b).start()

        return carry

    lax.fori_loop(0, ROUNDS, round_, 0)


N_LTILE = L // 8
N_BTILE = B // G
N_GROUPS = L * N_BTILE
GRP_PER_W = N_GROUPS // NW


def _detile_body(xt_hbm, out_hbm, sem):
    wid = lax.axis_index("s") * NC + lax.axis_index("c")

    def go(i, c):
        g = wid * GRP_PER_W + i
        l = g // N_BTILE
        bt = g % N_BTILE
        pltpu.make_async_copy(
            xt_hbm.at[l, pl.ds(bt * G, G)],
            out_hbm.at[pl.ds(l * B + bt * G, G)],
            sem,
        ).start()
        return c

    def drain(i, c):
        g = wid * GRP_PER_W + i
        l = g // N_BTILE
        bt = g % N_BTILE
        pltpu.make_async_copy(
            xt_hbm.at[l, pl.ds(bt * G, G)],
            out_hbm.at[pl.ds(l * B + bt * G, G)],
            sem,
        ).wait()
        return c

    lax.fori_loop(0, GRP_PER_W, go, 0)
    lax.fori_loop(0, GRP_PER_W, drain, 0)


def _detile(x_t):
    run = functools.partial(
        pl.kernel,
        mesh=plsc.VectorSubcoreMesh(core_axis_name="c", subcore_axis_name="s"),
        out_type=jax.ShapeDtypeStruct((TOK,), jnp.int32),
        scratch_types=[pltpu.SemaphoreType.DMA],
        compiler_params=pltpu.CompilerParams(use_tc_tiling_on_sc=True),
    )(_detile_body)
    return run(x_t)


def _gather(table, x_flat):
    run = functools.partial(
        pl.kernel,
        mesh=plsc.VectorSubcoreMesh(core_axis_name="c", subcore_axis_name="s"),
        out_type=jax.ShapeDtypeStruct((TOK, H), jnp.float32),
        scratch_types=[
            pltpu.VMEM((TOK_PER_W,), jnp.int32),
            pltpu.VMEM((NBUF, G, H), jnp.float32),
            pltpu.SemaphoreType.DMA((NBUF,)),
            pltpu.SemaphoreType.DMA((NBUF,)),
            pltpu.SemaphoreType.DMA,
        ],
        compiler_params=pltpu.CompilerParams(use_tc_tiling_on_sc=False),
    )(_body)
    return run(table, x_flat)


@jax.jit
def kernel(x, shared_weights):
    x_t = x.T
    raw = _gather(shared_weights, _detile(x_t))
    raw = raw.reshape(L, B, H)
    scale = jnp.where(x_t == 0, jnp.float32(0.0), jnp.float32(SCALE))
    out_t = raw * scale[..., None]
    return out_t.transpose(1, 0, 2)


# speedup vs baseline: 1.0030x; 1.0030x over previous
import functools

import jax
import jax.numpy as jnp
from jax import lax
from jax.experimental import pallas as pl
from jax.experimental.pallas import tpu as pltpu
from jax.experimental.pallas import tpu_sc as plsc

VOCAB_SIZE = 1000000
H = 64
B = 1024
L = 200
TOK = B * L
G = 128
SCALE = float(H) ** 0.5

_info = plsc.get_sparse_core_info()
NC = _info.num_cores
NS = _info.num_subcores
NW = NC * NS
ROWS_PER_W = B // NW
TOK_PER_W = ROWS_PER_W * L
GROUPS_PER_W = TOK_PER_W // G
NBUF = 5
ROUNDS = GROUPS_PER_W // NBUF


def _body(table_hbm, idx_hbm, out_hbm, idx_v, rows_v, gsem, osem, isem):
    wid = lax.axis_index("s") * NC + lax.axis_index("c")
    tok0 = wid * TOK_PER_W

    def idx_dma(i):
        g = wid * GRP_PER_W + i
        return pltpu.make_async_copy(
            idx_hbm.at[g // N_BTILE, pl.ds((g % N_BTILE) * G, G)],
            idx_v.at[pl.ds(i * G, G)],
            isem,
        )

    def start_idx(i, c):
        idx_dma(i).start()
        return c

    def wait_idx(i, c):
        idx_dma(i).wait()
        return c

    lax.fori_loop(0, GRP_PER_W, start_idx, 0)
    lax.fori_loop(0, GRP_PER_W, wait_idx, 0)

    def gather_of(g, b):
        return pltpu.make_async_copy(
            table_hbm.at[idx_v.at[pl.ds(g * G, G)]], rows_v.at[b], gsem.at[b]
        )

    def write_of(g, b):
        return pltpu.make_async_copy(
            rows_v.at[b], out_hbm.at[pl.ds(tok0 + g * G, G)], osem.at[b]
        )

    for b in range(NBUF):
        gather_of(b, b).start()

    def round_(o, carry):
        for b in range(NBUF):
            g = o * NBUF + b
            gather_of(g, b).wait()
            write_of(g, b).start()
        for b in range(NBUF):
            g = o * NBUF + b
            write_of(g, b).wait()

            @pl.when(o < ROUNDS - 1)
            def _():
                gather_of(g + NBUF, b).start()

        return carry

    lax.fori_loop(0, ROUNDS, round_, 0)


N_LTILE = L // 8
N_BTILE = B // G
N_GROUPS = L * N_BTILE
GRP_PER_W = N_GROUPS // NW


def _detile_body(xt_hbm, out_hbm, sem):
    wid = lax.axis_index("s") * NC + lax.axis_index("c")

    def go(i, c):
        g = wid * GRP_PER_W + i
        l = g // N_BTILE
        bt = g % N_BTILE
        pltpu.make_async_copy(
            xt_hbm.at[l, pl.ds(bt * G, G)],
            out_hbm.at[pl.ds(l * B + bt * G, G)],
            sem,
        ).start()
        return c

    def drain(i, c):
        g = wid * GRP_PER_W + i
        l = g // N_BTILE
        bt = g % N_BTILE
        pltpu.make_async_copy(
            xt_hbm.at[l, pl.ds(bt * G, G)],
            out_hbm.at[pl.ds(l * B + bt * G, G)],
            sem,
        ).wait()
        return c

    lax.fori_loop(0, GRP_PER_W, go, 0)
    lax.fori_loop(0, GRP_PER_W, drain, 0)


def _detile(x_t):
    run = functools.partial(
        pl.kernel,
        mesh=plsc.VectorSubcoreMesh(core_axis_name="c", subcore_axis_name="s"),
        out_type=jax.ShapeDtypeStruct((TOK,), jnp.int32),
        scratch_types=[pltpu.SemaphoreType.DMA],
        compiler_params=pltpu.CompilerParams(use_tc_tiling_on_sc=True),
    )(_detile_body)
    return run(x_t)


def _gather(table, x_t):
    run = functools.partial(
        pl.kernel,
        mesh=plsc.VectorSubcoreMesh(core_axis_name="c", subcore_axis_name="s"),
        out_type=jax.ShapeDtypeStruct((TOK, H), jnp.float32),
        scratch_types=[
            pltpu.VMEM((TOK_PER_W,), jnp.int32),
            pltpu.VMEM((NBUF, G, H), jnp.float32),
            pltpu.SemaphoreType.DMA((NBUF,)),
            pltpu.SemaphoreType.DMA((NBUF,)),
            pltpu.SemaphoreType.DMA,
        ],
        compiler_params=pltpu.CompilerParams(use_tc_tiling_on_sc=False),
    )(_body)
    return run(table, x_t)


@jax.jit
def kernel(x, shared_weights):
    x_t = x.T
    raw = _gather(shared_weights, x_t)
    raw = raw.reshape(L, B, H)
    scale = jnp.where(x_t == 0, jnp.float32(0.0), jnp.float32(SCALE))
    out_t = raw * scale[..., None]
    return out_t.transpose(1, 0, 2)
